# baseline (device time: 46517 ns/iter reference)
import jax
import jax.numpy as jnp
from jax import lax
from jax.experimental import pallas as pl
from jax.experimental.pallas import tpu as pltpu

N_DEV = 4
N_PIECES = 4


def kernel(x, w_mat):
    m, k_per = x.shape
    _, n = w_mat.shape
    m_per = m // N_DEV
    n_half = n // 2
    n_q = n_half // N_PIECES

    def body(x_ref, w_ref, out_ref,
             comm_r, comm_l, send_r, recv_r, send_l, recv_l):
        my = lax.axis_index("i")
        left = lax.rem(my + N_DEV - 1, N_DEV)
        right = lax.rem(my + 1, N_DEV)

        barrier_sem = pltpu.get_barrier_semaphore()
        for nbr in (left, right):
            pl.semaphore_signal(
                barrier_sem, inc=1,
                device_id=(nbr,), device_id_type=pl.DeviceIdType.MESH,
            )
        pl.semaphore_wait(barrier_sem, 2)

        def xs(j):
            return x_ref[pl.ds(j * m_per, m_per), :]

        def col_base(dir_r, piece):
            return (0 if dir_r else n_half) + piece * n_q

        def mk(dir_r, piece, src_slot, dst_slot):
            comm = comm_r if dir_r else comm_l
            ssem = send_r if dir_r else send_l
            rsem = recv_r if dir_r else recv_l
            tgt = right if dir_r else left
            return pltpu.make_async_remote_copy(
                src_ref=comm.at[src_slot, piece],
                dst_ref=comm.at[dst_slot, piece],
                send_sem=ssem.at[src_slot, piece],
                recv_sem=rsem.at[dst_slot, piece],
                device_id=(tgt,),
                device_id_type=pl.DeviceIdType.MESH,
            )

        PIECES = tuple(
            (dir_r, p) for p in range(N_PIECES) for dir_r in (True, False)
        )
        sends = []

        jr0 = lax.rem(my + N_DEV - 1, N_DEV)
        jl0 = lax.rem(my + 1, N_DEV)
        for dir_r, piece in PIECES:
            j = jr0 if dir_r else jl0
            comm = comm_r if dir_r else comm_l
            b = col_base(dir_r, piece)
            comm[0, piece] = jnp.dot(
                xs(j), w_ref[:, b:b + n_q],
                preferred_element_type=jnp.float32).astype(jnp.bfloat16)
            d = mk(dir_r, piece, 0, 1)
            d.start()
            sends.append(d)

        for t in range(1, N_DEV):
            jr = lax.rem(my + 2 * N_DEV - 1 - t, N_DEV)
            jl = lax.rem(my + 1 + t, N_DEV)
            p_r = jnp.dot(xs(jr), w_ref[:, :n_half],
                          preferred_element_type=jnp.float32)
            p_l = jnp.dot(xs(jl), w_ref[:, n_half:],
                          preferred_element_type=jnp.float32)

            for dir_r, piece in PIECES:
                comm = comm_r if dir_r else comm_l
                p = p_r if dir_r else p_l
                mk(dir_r, piece, t - 1, t).wait_recv()
                acc = (p[:, piece * n_q:(piece + 1) * n_q]
                       + comm[t, piece].astype(jnp.float32))
                if t < N_DEV - 1:
                    comm[t, piece] = acc.astype(jnp.bfloat16)
                    d = mk(dir_r, piece, t, t + 1)
                    d.start()
                    sends.append(d)
                else:
                    b = col_base(dir_r, piece)
                    out_ref[:, b:b + n_q] = acc * jax.nn.sigmoid(acc)

        for d in sends:
            d.wait_send()

    return pl.pallas_call(
        body,
        out_shape=jax.ShapeDtypeStruct((m_per, n), jnp.float32),
        in_specs=[
            pl.BlockSpec(memory_space=pltpu.VMEM),
            pl.BlockSpec(memory_space=pltpu.VMEM),
        ],
        out_specs=pl.BlockSpec(memory_space=pltpu.VMEM),
        scratch_shapes=[
            pltpu.VMEM((N_DEV, N_PIECES, m_per, n_q), jnp.bfloat16),
            pltpu.VMEM((N_DEV, N_PIECES, m_per, n_q), jnp.bfloat16),
            pltpu.SemaphoreType.DMA((N_DEV, N_PIECES)),
            pltpu.SemaphoreType.DMA((N_DEV, N_PIECES)),
            pltpu.SemaphoreType.DMA((N_DEV, N_PIECES)),
            pltpu.SemaphoreType.DMA((N_DEV, N_PIECES)),
        ],
        compiler_params=pltpu.CompilerParams(collective_id=0),
    )(x, w_mat)
